# Initial kernel scaffold; baseline (speedup 1.0000x reference)
import jax
import jax.numpy as jnp
from jax.experimental import pallas as pl


def kernel(scores, log_temperature):
    raise NotImplementedError("write your pallas kernel here")


# speedup vs baseline: 2.3239x; 2.3239x over previous
import functools

import jax
import jax.numpy as jnp
from jax.experimental import pallas as pl
from jax.experimental.pallas import tpu as pltpu

_K = 256.0
_B = 64
_T = 32768
_ROWS_PER_BLOCK = 8


def _tc_body(scale_ref, x_ref, o_ref):
    inv_temp = scale_ref[0]
    y = jax.nn.sigmoid(x_ref[...] * inv_temp)
    budget = jnp.clip(jnp.sum(y, axis=1, keepdims=True), 1e-6, None)
    y = y * jnp.minimum(_K / budget, 1.0)
    for d in (1, 2):
        shifted = pltpu.roll(y, shift=_T - d, axis=1)
        y = y * jnp.minimum(2.0 / (1.0 + y + shifted), 1.0)
    col = jax.lax.broadcasted_iota(jnp.int32, y.shape, 1)
    o_ref[...] = jnp.where(col == 0, 0.0, y)


@jax.jit
def kernel(scores, log_temperature):
    temp = jnp.clip(jnp.exp(log_temperature), 0.1, 10.0)
    inv_temp = (1.0 / temp).reshape(1).astype(jnp.float32)
    grid = (_B // _ROWS_PER_BLOCK,)
    return pl.pallas_call(
        _tc_body,
        grid=grid,
        in_specs=[
            pl.BlockSpec(memory_space=pltpu.SMEM),
            pl.BlockSpec((_ROWS_PER_BLOCK, _T), lambda i: (i, 0)),
        ],
        out_specs=pl.BlockSpec((_ROWS_PER_BLOCK, _T), lambda i: (i, 0)),
        out_shape=jax.ShapeDtypeStruct((_B, _T), jnp.float32),
        compiler_params=pltpu.CompilerParams(
            dimension_semantics=("arbitrary",),
        ),
    )(inv_temp, scores)
